# baseline (speedup 1.0000x reference)
import jax
import jax.numpy as jnp
from jax import lax
from jax.experimental import pallas as pl
from jax.experimental.pallas import tpu as pltpu
from jax.experimental.pallas import tpu_sc as plsc

_N = 10000
_E = 320000
_D = 128
_H = 128

_NC = 2
_NS = 16
_NW = _NC * _NS
_NPAD = 10240
_RPT = _NPAD // _NS

_KC = 128
_CROWS = 2560
_RW1 = _CROWS // _NW
_RW2 = _CROWS // _NS


def _mesh():
    return plsc.VectorSubcoreMesh(
        core_axis_name="c", subcore_axis_name="s",
        num_cores=_NC, num_subcores=_NS)


def _fill_ones(buf, nrows):
    one = jnp.ones((16,), jnp.float32)

    def fill(r, carry):
        for k in range(_D // 16):
            buf[r, pl.ds(k * 16, 16)] = one
        return carry

    lax.fori_loop(0, nrows, fill, 0)


def _deg_body(dst2, zeros, out, idst, ones_v, acc):
    c = lax.axis_index("c")
    s = lax.axis_index("s")
    wid = c * _NS + s
    r0 = s * _RPT
    pltpu.sync_copy(zeros.at[pl.ds(r0, _RPT)], acc.at[pl.ds(r0, _RPT)])
    pltpu.sync_copy(dst2.at[pl.ds(wid * _RW1, _RW1)], idst)
    _fill_ones(ones_v, _KC)
    plsc.subcore_barrier()

    def step(j, carry):
        pltpu.sync_copy(ones_v, acc.at[idst.at[j]], add=True)
        return carry

    lax.fori_loop(0, _RW1, step, 0)
    plsc.subcore_barrier()
    pltpu.sync_copy(acc.at[pl.ds(r0, _RPT)], out.at[c, pl.ds(r0, _RPT)])


def _sc_degree(dst2, zeros):
    f = pl.kernel(
        _deg_body,
        out_type=jax.ShapeDtypeStruct((_NC, _NPAD, _D), jnp.float32),
        mesh=_mesh(),
        scratch_types=[
            pltpu.VMEM((_RW1, _KC), jnp.int32),
            pltpu.VMEM((_KC, _D), jnp.float32),
            pltpu.VMEM_SHARED((_NPAD, _D), jnp.float32),
        ],
    )
    return f(dst2, zeros)


_BK = 16


def _pipe(z, src2, dst2, isrc, idst, rows0, rows1, acc, sem0, sem1,
          row0, nblocks):
    def g(j, rows, sem):
        pltpu.async_copy(z.at[isrc.at[j]], rows, sem)

    def w(j, rows, sem):
        pltpu.make_async_copy(z.at[isrc.at[j]], rows, sem).wait()

    def sc(j, rows):
        pltpu.sync_copy(rows, acc.at[idst.at[j]], add=True)

    def block(b, carry):
        r = pl.multiple_of(row0 + b * _BK, 8)
        pltpu.sync_copy(src2.at[pl.ds(r, _BK)], isrc)
        pltpu.sync_copy(dst2.at[pl.ds(r, _BK)], idst)
        g(0, rows0, sem0)

        @pl.loop(0, _BK - 2, step=2)
        def _body(j):
            g(j + 1, rows1, sem1)
            w(j, rows0, sem0)
            sc(j, rows0)
            g(j + 2, rows0, sem0)
            w(j + 1, rows1, sem1)
            sc(j + 1, rows1)

        g(_BK - 1, rows1, sem1)
        w(_BK - 2, rows0, sem0)
        sc(_BK - 2, rows0)
        w(_BK - 1, rows1, sem1)
        sc(_BK - 1, rows1)
        return carry

    lax.fori_loop(0, nblocks, block, 0)


def _prop1_body(z, src2, dst2, zeros, out,
                isrc, idst, rows0, rows1, acc, sem0, sem1):
    c = lax.axis_index("c")
    s = lax.axis_index("s")
    wid = c * _NS + s
    r0 = s * _RPT
    pltpu.sync_copy(zeros.at[pl.ds(r0, _RPT)], acc.at[pl.ds(r0, _RPT)])
    plsc.subcore_barrier()
    _pipe(z, src2, dst2, isrc, idst, rows0, rows1, acc, sem0, sem1,
          wid * _RW1, _RW1 // _BK)
    plsc.subcore_barrier()
    pltpu.sync_copy(acc.at[pl.ds(r0, _RPT)], out.at[c, pl.ds(r0, _RPT)])


def _sc_prop1(zarr, src2, dst2, zeros):
    f = pl.kernel(
        _prop1_body,
        out_type=jax.ShapeDtypeStruct((_NC, _NPAD, _D), jnp.float32),
        mesh=_mesh(),
        scratch_types=[
            pltpu.VMEM((_BK, _KC), jnp.int32),
            pltpu.VMEM((_BK, _KC), jnp.int32),
            pltpu.VMEM((_KC, _D), jnp.float32),
            pltpu.VMEM((_KC, _D), jnp.float32),
            pltpu.VMEM_SHARED((_NPAD, _D), jnp.float32),
            pltpu.SemaphoreType.DMA,
            pltpu.SemaphoreType.DMA,
        ],
    )
    return f(zarr, src2, dst2, zeros)


def _prop23_body(zu, zc, src2, dst2, zeros, out,
                 isrc, idst, rows0, rows1, acc, sem0, sem1):
    c = lax.axis_index("c")
    s = lax.axis_index("s")
    r0 = s * _RPT
    pltpu.sync_copy(zeros.at[pl.ds(r0, _RPT)], acc.at[pl.ds(r0, _RPT)])
    plsc.subcore_barrier()

    @pl.when(c == 0)
    def _u():
        _pipe(zu, src2, dst2, isrc, idst, rows0, rows1, acc, sem0, sem1,
              s * _RW2, _RW2 // _BK)

    @pl.when(c == 1)
    def _c():
        _pipe(zc, src2, dst2, isrc, idst, rows0, rows1, acc, sem0, sem1,
              s * _RW2, _RW2 // _BK)

    plsc.subcore_barrier()
    pltpu.sync_copy(acc.at[pl.ds(r0, _RPT)], out.at[c, pl.ds(r0, _RPT)])


def _sc_prop23(zu, zc, src2, dst2, zeros):
    f = pl.kernel(
        _prop23_body,
        out_type=jax.ShapeDtypeStruct((_NC, _NPAD, _D), jnp.float32),
        mesh=_mesh(),
        scratch_types=[
            pltpu.VMEM((_BK, _KC), jnp.int32),
            pltpu.VMEM((_BK, _KC), jnp.int32),
            pltpu.VMEM((_KC, _D), jnp.float32),
            pltpu.VMEM((_KC, _D), jnp.float32),
            pltpu.VMEM_SHARED((_NPAD, _D), jnp.float32),
            pltpu.SemaphoreType.DMA,
            pltpu.SemaphoreType.DMA,
        ],
    )
    return f(zu, zc, src2, dst2, zeros)


_R = 2000
_G = _N // _R


def _pre_body(dpart, x, dinv_o, z1_o):
    indeg = dpart[0][:, 0:1] + dpart[1][:, 0:1]
    dinv = lax.rsqrt(indeg + 1.0)
    dinv_o[...] = dinv
    z1_o[...] = x[...] * dinv


def _tc_pre(dpart, x):
    return pl.pallas_call(
        _pre_body,
        grid=(_G,),
        in_specs=[
            pl.BlockSpec((_NC, _R, _D), lambda i: (0, i, 0)),
            pl.BlockSpec((_R, _D), lambda i: (i, 0)),
        ],
        out_specs=[
            pl.BlockSpec((_R, 1), lambda i: (i, 0)),
            pl.BlockSpec((_R, _D), lambda i: (i, 0)),
        ],
        out_shape=[
            jax.ShapeDtypeStruct((_N, 1), jnp.float32),
            jax.ShapeDtypeStruct((_N, _D), jnp.float32),
        ],
    )(dpart, x)


def _wfold_body(W2u, W2c, b2u, b2c, uW, cW, ub, cb, Wfu_o, Wfc_o, bfu_o, bfc_o):
    Wfu_o[...] = jnp.dot(W2u[...], uW[...], preferred_element_type=jnp.float32)
    Wfc_o[...] = jnp.dot(W2c[...], cW[...], preferred_element_type=jnp.float32)
    bfu_o[...] = jnp.dot(b2u[...], uW[...], preferred_element_type=jnp.float32) + ub[...]
    bfc_o[...] = jnp.dot(b2c[...], cW[...], preferred_element_type=jnp.float32) + cb[...]


def _tc_wfold(W2u, W2c, b2u, b2c, uW, cW, ub, cb):
    return pl.pallas_call(
        _wfold_body,
        out_shape=[
            jax.ShapeDtypeStruct((3 * _H, _H), jnp.float32),
            jax.ShapeDtypeStruct((3 * _H, _H), jnp.float32),
            jax.ShapeDtypeStruct((1, _H), jnp.float32),
            jax.ShapeDtypeStruct((1, _H), jnp.float32),
        ],
    )(W2u, W2c, b2u, b2c, uW, cW, ub, cb)


def _mid_body(s1, z1, dinv, W1, b1, Wfu, Wfc, zu_o, zc_o):
    dv = dinv[...]
    q = (s1[0] + s1[1] + z1[...]) * dv
    h1 = jnp.dot(q, W1[...], preferred_element_type=jnp.float32) + b1[...]
    h1 = jnp.maximum(h1, 0.0)
    zu_o[...] = jnp.dot(h1, Wfu[...], preferred_element_type=jnp.float32) * dv
    zc_o[...] = jnp.dot(h1, Wfc[...], preferred_element_type=jnp.float32) * dv


def _tc_mid(s1, z1, dinv, W1, b1, Wfu, Wfc):
    return pl.pallas_call(
        _mid_body,
        grid=(_G,),
        in_specs=[
            pl.BlockSpec((_NC, _R, _D), lambda i: (0, i, 0)),
            pl.BlockSpec((_R, _D), lambda i: (i, 0)),
            pl.BlockSpec((_R, 1), lambda i: (i, 0)),
            pl.BlockSpec((_D, 3 * _H), lambda i: (0, 0)),
            pl.BlockSpec((1, 3 * _H), lambda i: (0, 0)),
            pl.BlockSpec((3 * _H, _H), lambda i: (0, 0)),
            pl.BlockSpec((3 * _H, _H), lambda i: (0, 0)),
        ],
        out_specs=[
            pl.BlockSpec((_R, _H), lambda i: (i, 0)),
            pl.BlockSpec((_R, _H), lambda i: (i, 0)),
        ],
        out_shape=[
            jax.ShapeDtypeStruct((_N, _H), jnp.float32),
            jax.ShapeDtypeStruct((_N, _H), jnp.float32),
        ],
    )(s1, z1, dinv, W1, b1, Wfu, Wfc)


def _post_body(s2, zu, zc, dinv, bfu, bfc, oW, ob, out_o, h_o):
    dv = dinv[...]
    pu = (s2[0] + zu[...]) * dv + bfu[...]
    pc = (s2[1] + zc[...]) * dv + bfc[...]
    u = jax.nn.sigmoid(pu)
    cg = jnp.tanh(pc)
    h = (1.0 - u) * cg
    h_o[...] = h
    out_o[...] = jnp.dot(h, oW[...], preferred_element_type=jnp.float32) + ob[...]


def _tc_post(s2, zu, zc, dinv, bfu, bfc, oW, ob):
    return pl.pallas_call(
        _post_body,
        grid=(_G,),
        in_specs=[
            pl.BlockSpec((_NC, _R, _H), lambda i: (0, i, 0)),
            pl.BlockSpec((_R, _H), lambda i: (i, 0)),
            pl.BlockSpec((_R, _H), lambda i: (i, 0)),
            pl.BlockSpec((_R, 1), lambda i: (i, 0)),
            pl.BlockSpec((1, _H), lambda i: (0, 0)),
            pl.BlockSpec((1, _H), lambda i: (0, 0)),
            pl.BlockSpec((_H, 1), lambda i: (0, 0)),
            pl.BlockSpec((1, 1), lambda i: (0, 0)),
        ],
        out_specs=[
            pl.BlockSpec((_R, 1), lambda i: (i, 0)),
            pl.BlockSpec((_R, _H), lambda i: (i, 0)),
        ],
        out_shape=[
            jax.ShapeDtypeStruct((_N, 1), jnp.float32),
            jax.ShapeDtypeStruct((_N, _H), jnp.float32),
        ],
    )(s2, zu, zc, dinv, bfu, bfc, oW, ob)


def kernel(x, edge_index, gcn_W1, gcn_b1, gcn_W2, gcn_b2,
           u_W, u_b, r_W, r_b, c_W, c_b, out_W, out_b):
    f32 = jnp.float32
    zeros128 = jnp.zeros((_NPAD, _D), f32)

    padn = _CROWS * _KC - _E
    src2 = jnp.concatenate(
        [edge_index[0], jnp.zeros((padn,), jnp.int32)]).reshape(_CROWS, _KC)
    dst2 = jnp.concatenate(
        [edge_index[1], jnp.full((padn,), _NPAD - 1, jnp.int32)]).reshape(_CROWS, _KC)

    dpart = _sc_degree(dst2, zeros128)
    dinv, z1 = _tc_pre(dpart, x)
    s1 = _sc_prop1(z1, src2, dst2, zeros128)

    Wfu, Wfc, bfu, bfc = _tc_wfold(
        gcn_W2[:, :_H], gcn_W2[:, 2 * _H:],
        gcn_b2[:_H].reshape(1, _H), gcn_b2[2 * _H:].reshape(1, _H),
        u_W[:_H], c_W[:_H],
        u_b.reshape(1, _H), c_b.reshape(1, _H),
    )
    zu, zc = _tc_mid(s1, z1, dinv, gcn_W1, gcn_b1.reshape(1, 3 * _H), Wfu, Wfc)

    s2 = _sc_prop23(zu, zc, src2, dst2, zeros128)

    out, h = _tc_post(s2, zu, zc, dinv, bfu, bfc,
                      out_W, out_b.reshape(1, 1))
    return out, h
